# Initial kernel scaffold; baseline (speedup 1.0000x reference)
import jax
import jax.numpy as jnp
from jax.experimental import pallas as pl


def kernel(inputs, offsets):
    raise NotImplementedError("write your pallas kernel here")


# speedup vs baseline: 6.7777x; 6.7777x over previous
import functools

import jax
import jax.numpy as jnp
from jax import lax
from jax.experimental import pallas as pl
from jax.experimental.pallas import tpu as pltpu
from jax.experimental.pallas import tpu_sc as plsc

N_TOK = 32768
D = 256
NSEG = 16
L = 16
NW = 32
BS = 128
BS_LOG2 = 7
NBLK = N_TOK // BS
BPW = NBLK // NW
RW = N_TOK // NW
DC = D // L
H = D // 2
HC = H // L
EW = BS + 8

_MESH = plsc.VectorSubcoreMesh(core_axis_name="c", subcore_axis_name="s")


@functools.partial(
    pl.kernel,
    mesh=_MESH,
    out_type=jax.ShapeDtypeStruct((2, NBLK, H), jnp.float32),
    scratch_types=[
        pltpu.VMEM((2, BS, D), jnp.float32),
        pltpu.VMEM((BPW, H), jnp.float32),
        pltpu.VMEM((BPW, H), jnp.float32),
        pltpu.SemaphoreType.DMA,
        pltpu.SemaphoreType.DMA,
    ],
)
def _block_sums(feats_hbm, blocks_hbm, buf, bsum_lo, bsum_hi, sem0, sem1):
    wid = lax.axis_index("c") * 16 + lax.axis_index("s")
    base_row = wid * RW
    sems = (sem0, sem1)
    copies = [None] * BPW

    def start(c):
        copies[c] = pltpu.async_copy(
            feats_hbm.at[pl.ds(base_row + c * BS, BS), :],
            buf.at[c % 2],
            sems[c % 2],
        )

    start(0)
    for c in range(BPW):
        if c + 1 < BPW:
            start(c + 1)
        copies[c].wait()
        b = c % 2

        def body(r, acc, _b=b):
            return tuple(
                acc[k] + buf[_b, r, pl.ds(k * L, L)] for k in range(DC)
            )

        acc = lax.fori_loop(
            0, BS, body, tuple(jnp.zeros((L,), jnp.float32) for _ in range(DC))
        )
        for k in range(HC):
            bsum_lo[c, pl.ds(k * L, L)] = acc[k]
            bsum_hi[c, pl.ds(k * L, L)] = acc[HC + k]
    pltpu.sync_copy(bsum_lo, blocks_hbm.at[0, pl.ds(wid * BPW, BPW), :])
    pltpu.sync_copy(bsum_hi, blocks_hbm.at[1, pl.ds(wid * BPW, BPW), :])


@functools.partial(
    pl.kernel,
    mesh=_MESH,
    out_type=jax.ShapeDtypeStruct((NSEG * D,), jnp.float32),
    scratch_types=[
        pltpu.VMEM((NBLK, H), jnp.float32),
        pltpu.VMEM((EW, H), jnp.float32),
        pltpu.VMEM((BS, H), jnp.float32),
        pltpu.VMEM((2 * L,), jnp.int32),
        pltpu.VMEM((H,), jnp.float32),
    ],
)
def _seg_means(feats_hbm, offs_hbm, blocks_hbm, out_hbm, blk, e1, e2, offs_v, out_v):
    wid = lax.axis_index("c") * 16 + lax.axis_index("s")
    j = wid % NSEG
    h = wid // NSEG
    hoff = h * H

    pltpu.sync_copy(offs_hbm, offs_v.at[pl.ds(0, NSEG + 1)])
    lo = offs_v[pl.ds(j, L)][0]
    hi = offs_v[pl.ds(j + 1, L)][0]

    fb = (lo + BS - 1) >> BS_LOG2
    lb = hi >> BS_LOG2
    a_row = fb << BS_LOG2
    b_row = lb << BS_LOG2
    e1_hi = jnp.minimum(a_row, hi)
    e2_lo = jnp.maximum(b_row, e1_hi)
    base1 = pl.multiple_of(jnp.minimum(lo & ~7, N_TOK - EW), 8)
    b_row_al = pl.multiple_of(b_row, BS)

    pltpu.sync_copy(blocks_hbm.at[h], blk)
    pltpu.sync_copy(feats_hbm.at[pl.ds(base1, EW), pl.ds(hoff, H)], e1)
    pltpu.sync_copy(feats_hbm.at[pl.ds(b_row_al, BS), pl.ds(hoff, H)], e2)

    def mk_body(ref):
        def body(r, acc):
            return tuple(acc[k] + ref[r, pl.ds(k * L, L)] for k in range(HC))
        return body

    zeros = tuple(jnp.zeros((L,), jnp.float32) for _ in range(HC))
    acc_b = lax.fori_loop(fb, jnp.maximum(fb, lb), mk_body(blk), zeros)
    acc_1 = lax.fori_loop(lo - base1, e1_hi - base1, mk_body(e1), zeros)
    acc_2 = lax.fori_loop(e2_lo - b_row, hi - b_row, mk_body(e2), zeros)

    cnt = jnp.maximum(hi - lo, 1).astype(jnp.float32)
    inv_n = jnp.ones((L,), jnp.float32) / jnp.full((L,), cnt)
    for k in range(HC):
        out_v[pl.ds(k * L, L)] = (acc_b[k] + acc_1[k] + acc_2[k]) * inv_n
    pltpu.sync_copy(out_v, out_hbm.at[pl.ds(pl.multiple_of(j * D + hoff, H), H)])


def kernel(inputs, offsets):
    feats = inputs
    offs = offsets.astype(jnp.int32)
    blocks = _block_sums(feats)
    return _seg_means(feats, offs, blocks).reshape(NSEG, D)


# speedup vs baseline: 6.9984x; 1.0326x over previous
import functools

import jax
import jax.numpy as jnp
from jax import lax
from jax.experimental import pallas as pl
from jax.experimental.pallas import tpu as pltpu
from jax.experimental.pallas import tpu_sc as plsc

N_TOK = 32768
D = 256
NSEG = 16
L = 16
NW = 32
BS = 128
BS_LOG2 = 7
NBLK = N_TOK // BS
BPW = NBLK // NW
RW = N_TOK // NW
DC = D // L
H = D // 2
HC = H // L
EW = BS + 8

_MESH = plsc.VectorSubcoreMesh(core_axis_name="c", subcore_axis_name="s")


@functools.partial(
    pl.kernel,
    mesh=_MESH,
    out_type=jax.ShapeDtypeStruct((2, NBLK, H), jnp.float32),
    scratch_types=[
        pltpu.VMEM((2, BS, D), jnp.float32),
        pltpu.VMEM((BPW, H), jnp.float32),
        pltpu.VMEM((BPW, H), jnp.float32),
        pltpu.VMEM((2 * L,), jnp.int32),
        pltpu.SemaphoreType.DMA,
        pltpu.SemaphoreType.DMA,
    ],
)
def _block_sums(feats_hbm, offs_hbm, blocks_hbm, buf, bsum_lo, bsum_hi,
                offs_v, sem0, sem1):
    wid = lax.axis_index("c") * 16 + lax.axis_index("s")
    base_row = wid * RW
    sems = (sem0, sem1)
    copies = [None] * BPW

    pltpu.sync_copy(offs_hbm, offs_v.at[pl.ds(0, NSEG + 1)])
    first = offs_v[pl.ds(0, L)][0]
    last = offs_v[pl.ds(NSEG, L)][0]

    def live(c):
        return (base_row + c * BS + BS > first) & (base_row + c * BS < last)

    def start(c):
        wid_i = lax.axis_index("c") * 16 + lax.axis_index("s")
        copies[c] = pltpu.async_copy(
            feats_hbm.at[pl.ds(wid_i * RW + c * BS, BS), :],
            buf.at[c % 2],
            sems[c % 2],
        )

    @pl.when(live(0))
    def _():
        start(0)

    for c in range(BPW):
        if c + 1 < BPW:
            @pl.when(live(c + 1))
            def _(_c=c + 1):
                start(_c)

        @pl.when(live(c))
        def _(_c=c):
            b = _c % 2
            pltpu.make_async_copy(
                feats_hbm.at[pl.ds(0, BS), :], buf.at[b], sems[b]
            ).wait()

            def body(r, acc):
                return tuple(
                    acc[k] + buf[b, r, pl.ds(k * L, L)] for k in range(DC)
                )

            acc = lax.fori_loop(
                0, BS, body,
                tuple(jnp.zeros((L,), jnp.float32) for _ in range(DC)),
            )
            for k in range(HC):
                bsum_lo[_c, pl.ds(k * L, L)] = acc[k]
                bsum_hi[_c, pl.ds(k * L, L)] = acc[HC + k]
    pltpu.sync_copy(bsum_lo, blocks_hbm.at[0, pl.ds(wid * BPW, BPW), :])
    pltpu.sync_copy(bsum_hi, blocks_hbm.at[1, pl.ds(wid * BPW, BPW), :])


@functools.partial(
    pl.kernel,
    mesh=_MESH,
    out_type=jax.ShapeDtypeStruct((NSEG * D,), jnp.float32),
    scratch_types=[
        pltpu.VMEM((NBLK, H), jnp.float32),
        pltpu.VMEM((EW, H), jnp.float32),
        pltpu.VMEM((BS, H), jnp.float32),
        pltpu.VMEM((2 * L,), jnp.int32),
        pltpu.VMEM((H,), jnp.float32),
        pltpu.SemaphoreType.DMA,
        pltpu.SemaphoreType.DMA,
        pltpu.SemaphoreType.DMA,
    ],
)
def _seg_means(feats_hbm, offs_hbm, blocks_hbm, out_hbm, blk, e1, e2, offs_v,
               out_v, sem_b, sem_1, sem_2):
    wid = lax.axis_index("c") * 16 + lax.axis_index("s")
    j = wid % NSEG
    h = wid // NSEG
    hoff = h * H

    cp_b = pltpu.async_copy(blocks_hbm.at[h], blk, sem_b)
    pltpu.sync_copy(offs_hbm, offs_v.at[pl.ds(0, NSEG + 1)])
    lo = offs_v[pl.ds(j, L)][0]
    hi = offs_v[pl.ds(j + 1, L)][0]

    fb = (lo + BS - 1) >> BS_LOG2
    lb = hi >> BS_LOG2
    a_row = fb << BS_LOG2
    b_row = lb << BS_LOG2
    e1_hi = jnp.minimum(a_row, hi)
    e2_lo = jnp.maximum(b_row, e1_hi)
    base1 = pl.multiple_of(jnp.minimum(lo & ~7, N_TOK - EW), 8)
    b_row_al = pl.multiple_of(b_row, BS)

    cp_1 = pltpu.async_copy(feats_hbm.at[pl.ds(base1, EW), pl.ds(hoff, H)], e1, sem_1)
    cp_2 = pltpu.async_copy(feats_hbm.at[pl.ds(b_row_al, BS), pl.ds(hoff, H)], e2, sem_2)
    cp_b.wait()
    cp_1.wait()
    cp_2.wait()

    def mk_body(ref):
        def body(r, acc):
            return tuple(acc[k] + ref[r, pl.ds(k * L, L)] for k in range(HC))
        return body

    zeros = tuple(jnp.zeros((L,), jnp.float32) for _ in range(HC))
    acc_b = lax.fori_loop(fb, jnp.maximum(fb, lb), mk_body(blk), zeros)
    acc_1 = lax.fori_loop(lo - base1, e1_hi - base1, mk_body(e1), zeros)
    acc_2 = lax.fori_loop(e2_lo - b_row, hi - b_row, mk_body(e2), zeros)

    cnt = jnp.maximum(hi - lo, 1).astype(jnp.float32)
    inv_n = jnp.ones((L,), jnp.float32) / jnp.full((L,), cnt)
    for k in range(HC):
        out_v[pl.ds(k * L, L)] = (acc_b[k] + acc_1[k] + acc_2[k]) * inv_n
    pltpu.sync_copy(out_v, out_hbm.at[pl.ds(pl.multiple_of(j * D + hoff, H), H)])


def kernel(inputs, offsets):
    feats = inputs
    offs = offsets.astype(jnp.int32)
    blocks = _block_sums(feats, offs)
    return _seg_means(feats, offs, blocks).reshape(NSEG, D)


# speedup vs baseline: 8.2131x; 1.1736x over previous
import functools

import jax
import jax.numpy as jnp
from jax import lax
from jax.experimental import pallas as pl
from jax.experimental.pallas import tpu as pltpu
from jax.experimental.pallas import tpu_sc as plsc

N_TOK = 32768
D = 256
NSEG = 16
L = 16
NW = 32
BS = 128
BS_LOG2 = 7
NBLK = N_TOK // BS
BPW = NBLK // NW
RW = N_TOK // NW
DC = D // L
H = D // 2
HC = H // L
EW = BS + 8

_MESH = plsc.VectorSubcoreMesh(core_axis_name="c", subcore_axis_name="s")


@functools.partial(
    pl.kernel,
    mesh=_MESH,
    out_type=jax.ShapeDtypeStruct((2, NBLK, H), jnp.float32),
    scratch_types=[
        pltpu.VMEM((2, BS, D), jnp.float32),
        pltpu.VMEM((BPW, H), jnp.float32),
        pltpu.VMEM((BPW, H), jnp.float32),
        pltpu.VMEM((2 * L,), jnp.int32),
        pltpu.SemaphoreType.DMA,
        pltpu.SemaphoreType.DMA,
    ],
)
def _block_sums(feats_hbm, offs_hbm, blocks_hbm, buf, bsum_lo, bsum_hi,
                offs_v, sem0, sem1):
    wid = lax.axis_index("c") * 16 + lax.axis_index("s")
    base_row = wid * RW
    sems = (sem0, sem1)
    copies = [None] * BPW

    pltpu.sync_copy(offs_hbm, offs_v.at[pl.ds(0, NSEG + 1)])
    first = offs_v[pl.ds(0, L)][0]
    last = offs_v[pl.ds(NSEG, L)][0]

    def live(c):
        return (base_row + c * BS + BS > first) & (base_row + c * BS < last)

    def start(c):
        wid_i = lax.axis_index("c") * 16 + lax.axis_index("s")
        copies[c] = pltpu.async_copy(
            feats_hbm.at[pl.ds(wid_i * RW + c * BS, BS), :],
            buf.at[c % 2],
            sems[c % 2],
        )

    @pl.when(live(0))
    def _():
        start(0)

    for c in range(BPW):
        if c + 1 < BPW:
            @pl.when(live(c + 1))
            def _(_c=c + 1):
                start(_c)

        @pl.when(live(c))
        def _(_c=c):
            b = _c % 2
            pltpu.make_async_copy(
                feats_hbm.at[pl.ds(0, BS), :], buf.at[b], sems[b]
            ).wait()

            def body(r, acc):
                return tuple(
                    acc[k] + buf[b, r, pl.ds(k * L, L)] for k in range(DC)
                )

            acc = lax.fori_loop(
                0, BS, body,
                tuple(jnp.zeros((L,), jnp.float32) for _ in range(DC)),
            )
            for k in range(HC):
                bsum_lo[_c, pl.ds(k * L, L)] = acc[k]
                bsum_hi[_c, pl.ds(k * L, L)] = acc[HC + k]
    pltpu.sync_copy(bsum_lo, blocks_hbm.at[0, pl.ds(wid * BPW, BPW), :])
    pltpu.sync_copy(bsum_hi, blocks_hbm.at[1, pl.ds(wid * BPW, BPW), :])


@functools.partial(
    pl.kernel,
    mesh=_MESH,
    out_type=jax.ShapeDtypeStruct((NSEG * D,), jnp.float32),
    scratch_types=[
        pltpu.VMEM((NBLK, H), jnp.float32),
        pltpu.VMEM((EW, H), jnp.float32),
        pltpu.VMEM((BS, H), jnp.float32),
        pltpu.VMEM((2 * L,), jnp.int32),
        pltpu.VMEM((H,), jnp.float32),
        pltpu.SemaphoreType.DMA,
        pltpu.SemaphoreType.DMA,
        pltpu.SemaphoreType.DMA,
    ],
)
def _seg_means(feats_hbm, offs_hbm, blocks_hbm, out_hbm, blk, e1, e2, offs_v,
               out_v, sem_b, sem_1, sem_2):
    wid = lax.axis_index("c") * 16 + lax.axis_index("s")
    j = wid % NSEG
    h = wid // NSEG
    hoff = h * H

    cp_b = pltpu.async_copy(blocks_hbm.at[h], blk, sem_b)
    pltpu.sync_copy(offs_hbm, offs_v.at[pl.ds(0, NSEG + 1)])
    lo = offs_v[pl.ds(j, L)][0]
    hi = offs_v[pl.ds(j + 1, L)][0]

    fb = (lo + BS - 1) >> BS_LOG2
    lb = hi >> BS_LOG2
    a_row = fb << BS_LOG2
    b_row = lb << BS_LOG2
    e1_hi = jnp.minimum(a_row, hi)
    e2_lo = jnp.maximum(b_row, e1_hi)
    base1 = pl.multiple_of(jnp.minimum(lo & ~7, N_TOK - EW), 8)
    b_row_al = pl.multiple_of(b_row, BS)

    cp_1 = pltpu.async_copy(feats_hbm.at[pl.ds(base1, EW), pl.ds(hoff, H)], e1, sem_1)
    cp_2 = pltpu.async_copy(feats_hbm.at[pl.ds(b_row_al, BS), pl.ds(hoff, H)], e2, sem_2)
    cp_b.wait()
    cp_1.wait()
    cp_2.wait()

    def mk_body(ref):
        def body(r, acc):
            return tuple(acc[k] + ref[r, pl.ds(k * L, L)] for k in range(HC))
        return body

    zeros = tuple(jnp.zeros((L,), jnp.float32) for _ in range(HC))
    acc_b = lax.fori_loop(fb, jnp.maximum(fb, lb), mk_body(blk), zeros)
    acc_1 = lax.fori_loop(lo - base1, e1_hi - base1, mk_body(e1), zeros)
    acc_2 = lax.fori_loop(e2_lo - b_row, hi - b_row, mk_body(e2), zeros)

    cnt = jnp.maximum(hi - lo, 1).astype(jnp.float32)
    inv_n = jnp.ones((L,), jnp.float32) / jnp.full((L,), cnt)
    for k in range(HC):
        out_v[pl.ds(k * L, L)] = (acc_b[k] + acc_1[k] + acc_2[k]) * inv_n
    pltpu.sync_copy(out_v, out_hbm.at[pl.ds(pl.multiple_of(j * D + hoff, H), H)])


def _combine_tc_body(starts_smem, ends_smem, blocks_ref, feats_ref, out_ref,
                     lead_buf, trail_buf, lead_sem, trail_sem):
    lead_cp = [None] * NSEG
    trail_cp = [None] * NSEG
    for j in range(NSEG):
        lo = starts_smem[j]
        hi = ends_smem[j]
        lead_base = pl.multiple_of(jnp.minimum(lo & ~7, N_TOK - EW), 8)
        trail_base = pl.multiple_of((hi >> BS_LOG2) << BS_LOG2, BS)
        lead_cp[j] = pltpu.make_async_copy(
            feats_ref.at[pl.ds(lead_base, EW), :], lead_buf.at[j], lead_sem.at[j])
        trail_cp[j] = pltpu.make_async_copy(
            feats_ref.at[pl.ds(trail_base, BS), :], trail_buf.at[j], trail_sem.at[j])
        lead_cp[j].start()
        trail_cp[j].start()

    blk_full = jnp.concatenate([blocks_ref[0], blocks_ref[1]], axis=1)
    biota = lax.broadcasted_iota(jnp.int32, (1, NBLK), 1)
    rows = []
    for j in range(NSEG):
        fb = (starts_smem[j] + BS - 1) >> BS_LOG2
        lb = ends_smem[j] >> BS_LOG2
        rows.append(jnp.where((biota >= fb) & (biota < lb), 1.0, 0.0))
    mask = jnp.concatenate(rows, axis=0)
    part = jax.lax.dot_general(
        mask, blk_full, (((1,), (0,)), ((), ())),
        precision=jax.lax.Precision.HIGHEST,
        preferred_element_type=jnp.float32)

    riota_l = lax.broadcasted_iota(jnp.int32, (EW, 1), 0)
    riota_t = lax.broadcasted_iota(jnp.int32, (BS, 1), 0)
    for j in range(NSEG):
        lo = starts_smem[j]
        hi = ends_smem[j]
        lead_base = jnp.minimum(lo & ~7, N_TOK - EW)
        a_row = ((lo + BS - 1) >> BS_LOG2) << BS_LOG2
        b_row = (hi >> BS_LOG2) << BS_LOG2
        e1_hi = jnp.minimum(a_row, hi)
        e2_lo = jnp.maximum(b_row, e1_hi)
        lead_cp[j].wait()
        trail_cp[j].wait()
        ml = (riota_l >= lo - lead_base) & (riota_l < e1_hi - lead_base)
        mt = (riota_t >= e2_lo - b_row) & (riota_t < hi - b_row)
        lead_sum = jnp.sum(jnp.where(ml, lead_buf[j], 0.0), axis=0)
        trail_sum = jnp.sum(jnp.where(mt, trail_buf[j], 0.0), axis=0)
        inv_n = 1.0 / jnp.maximum(hi - lo, 1).astype(jnp.float32)
        out_ref[j, :] = (part[j, :] + lead_sum + trail_sum) * inv_n


def _combine_tc(starts, ends, blocks, feats):
    return pl.pallas_call(
        _combine_tc_body,
        out_shape=jax.ShapeDtypeStruct((NSEG, D), jnp.float32),
        in_specs=[
            pl.BlockSpec(memory_space=pltpu.SMEM),
            pl.BlockSpec(memory_space=pltpu.SMEM),
            pl.BlockSpec(memory_space=pltpu.VMEM),
            pl.BlockSpec(memory_space=pltpu.HBM),
        ],
        out_specs=pl.BlockSpec(memory_space=pltpu.VMEM),
        scratch_shapes=[
            pltpu.VMEM((NSEG, EW, D), jnp.float32),
            pltpu.VMEM((NSEG, BS, D), jnp.float32),
            pltpu.SemaphoreType.DMA((NSEG,)),
            pltpu.SemaphoreType.DMA((NSEG,)),
        ],
    )(starts, ends, blocks, feats)


def kernel(inputs, offsets):
    feats = inputs
    offs = offsets.astype(jnp.int32)
    blocks = _block_sums(feats, offs)
    return _combine_tc(offs[:NSEG], offs[1:], blocks, feats)


# speedup vs baseline: 8.6756x; 1.0563x over previous
import functools

import jax
import jax.numpy as jnp
from jax import lax
from jax.experimental import pallas as pl
from jax.experimental.pallas import tpu as pltpu
from jax.experimental.pallas import tpu_sc as plsc

N_TOK = 32768
D = 256
NSEG = 16
L = 16
NW = 32
BS = 128
BS_LOG2 = 7
NBLK = N_TOK // BS
BPW = NBLK // NW
RW = N_TOK // NW
DC = D // L
H = D // 2
HC = H // L
EW = BS + 8

_MESH = plsc.VectorSubcoreMesh(core_axis_name="c", subcore_axis_name="s")


@functools.partial(
    pl.kernel,
    mesh=_MESH,
    out_type=jax.ShapeDtypeStruct((2, NBLK, H), jnp.float32),
    scratch_types=[
        pltpu.VMEM((2, BS, D), jnp.float32),
        pltpu.VMEM((BPW, H), jnp.float32),
        pltpu.VMEM((BPW, H), jnp.float32),
        pltpu.VMEM((2 * L,), jnp.int32),
        pltpu.SemaphoreType.DMA,
        pltpu.SemaphoreType.DMA,
    ],
)
def _block_sums(feats_hbm, offs_hbm, blocks_hbm, buf, bsum_lo, bsum_hi,
                offs_v, sem0, sem1):
    wid = lax.axis_index("c") * 16 + lax.axis_index("s")
    base_row = wid * RW
    sems = (sem0, sem1)
    copies = [None] * BPW

    pltpu.sync_copy(offs_hbm, offs_v.at[pl.ds(0, NSEG + 1)])
    first = offs_v[pl.ds(0, L)][0]
    last = offs_v[pl.ds(NSEG, L)][0]

    def live(c):
        return (base_row + c * BS + BS > first) & (base_row + c * BS < last)

    def start(c):
        wid_i = lax.axis_index("c") * 16 + lax.axis_index("s")
        copies[c] = pltpu.async_copy(
            feats_hbm.at[pl.ds(wid_i * RW + c * BS, BS), :],
            buf.at[c % 2],
            sems[c % 2],
        )

    @pl.when(live(0))
    def _():
        start(0)

    for c in range(BPW):
        if c + 1 < BPW:
            @pl.when(live(c + 1))
            def _(_c=c + 1):
                start(_c)

        @pl.when(live(c))
        def _(_c=c):
            b = _c % 2
            pltpu.make_async_copy(
                feats_hbm.at[pl.ds(0, BS), :], buf.at[b], sems[b]
            ).wait()

            def body(r, acc):
                return tuple(
                    acc[k] + buf[b, r, pl.ds(k * L, L)] for k in range(DC)
                )

            acc = lax.fori_loop(
                0, BS, body,
                tuple(jnp.zeros((L,), jnp.float32) for _ in range(DC)),
            )
            for k in range(HC):
                bsum_lo[_c, pl.ds(k * L, L)] = acc[k]
                bsum_hi[_c, pl.ds(k * L, L)] = acc[HC + k]
    pltpu.sync_copy(bsum_lo, blocks_hbm.at[0, pl.ds(wid * BPW, BPW), :])
    pltpu.sync_copy(bsum_hi, blocks_hbm.at[1, pl.ds(wid * BPW, BPW), :])


@functools.partial(
    pl.kernel,
    mesh=_MESH,
    out_type=jax.ShapeDtypeStruct((NSEG * D,), jnp.float32),
    scratch_types=[
        pltpu.VMEM((NBLK, H), jnp.float32),
        pltpu.VMEM((EW, H), jnp.float32),
        pltpu.VMEM((BS, H), jnp.float32),
        pltpu.VMEM((2 * L,), jnp.int32),
        pltpu.VMEM((H,), jnp.float32),
        pltpu.SemaphoreType.DMA,
        pltpu.SemaphoreType.DMA,
        pltpu.SemaphoreType.DMA,
    ],
)
def _seg_means(feats_hbm, offs_hbm, blocks_hbm, out_hbm, blk, e1, e2, offs_v,
               out_v, sem_b, sem_1, sem_2):
    wid = lax.axis_index("c") * 16 + lax.axis_index("s")
    j = wid % NSEG
    h = wid // NSEG
    hoff = h * H

    cp_b = pltpu.async_copy(blocks_hbm.at[h], blk, sem_b)
    pltpu.sync_copy(offs_hbm, offs_v.at[pl.ds(0, NSEG + 1)])
    lo = offs_v[pl.ds(j, L)][0]
    hi = offs_v[pl.ds(j + 1, L)][0]

    fb = (lo + BS - 1) >> BS_LOG2
    lb = hi >> BS_LOG2
    a_row = fb << BS_LOG2
    b_row = lb << BS_LOG2
    e1_hi = jnp.minimum(a_row, hi)
    e2_lo = jnp.maximum(b_row, e1_hi)
    base1 = pl.multiple_of(jnp.minimum(lo & ~7, N_TOK - EW), 8)
    b_row_al = pl.multiple_of(b_row, BS)

    cp_1 = pltpu.async_copy(feats_hbm.at[pl.ds(base1, EW), pl.ds(hoff, H)], e1, sem_1)
    cp_2 = pltpu.async_copy(feats_hbm.at[pl.ds(b_row_al, BS), pl.ds(hoff, H)], e2, sem_2)
    cp_b.wait()
    cp_1.wait()
    cp_2.wait()

    def mk_body(ref):
        def body(r, acc):
            return tuple(acc[k] + ref[r, pl.ds(k * L, L)] for k in range(HC))
        return body

    zeros = tuple(jnp.zeros((L,), jnp.float32) for _ in range(HC))
    acc_b = lax.fori_loop(fb, jnp.maximum(fb, lb), mk_body(blk), zeros)
    acc_1 = lax.fori_loop(lo - base1, e1_hi - base1, mk_body(e1), zeros)
    acc_2 = lax.fori_loop(e2_lo - b_row, hi - b_row, mk_body(e2), zeros)

    cnt = jnp.maximum(hi - lo, 1).astype(jnp.float32)
    inv_n = jnp.ones((L,), jnp.float32) / jnp.full((L,), cnt)
    for k in range(HC):
        out_v[pl.ds(k * L, L)] = (acc_b[k] + acc_1[k] + acc_2[k]) * inv_n
    pltpu.sync_copy(out_v, out_hbm.at[pl.ds(pl.multiple_of(j * D + hoff, H), H)])


def _edges_tc_body(starts_smem, ends_smem, feats_ref, out_ref,
                   lead_buf, trail_buf, lead_sem, trail_sem):
    lead_cp = [None] * NSEG
    trail_cp = [None] * NSEG
    for j in range(NSEG):
        lo = starts_smem[j]
        hi = ends_smem[j]
        lead_base = pl.multiple_of(jnp.minimum(lo & ~7, N_TOK - EW), 8)
        trail_base = pl.multiple_of((hi >> BS_LOG2) << BS_LOG2, BS)
        lead_cp[j] = pltpu.make_async_copy(
            feats_ref.at[pl.ds(lead_base, EW), :], lead_buf.at[j], lead_sem.at[j])
        trail_cp[j] = pltpu.make_async_copy(
            feats_ref.at[pl.ds(trail_base, BS), :], trail_buf.at[j], trail_sem.at[j])
        lead_cp[j].start()
        trail_cp[j].start()

    riota_l = lax.broadcasted_iota(jnp.int32, (EW, 1), 0)
    riota_t = lax.broadcasted_iota(jnp.int32, (BS, 1), 0)
    for j in range(NSEG):
        lo = starts_smem[j]
        hi = ends_smem[j]
        lead_base = jnp.minimum(lo & ~7, N_TOK - EW)
        a_row = ((lo + BS - 1) >> BS_LOG2) << BS_LOG2
        b_row = (hi >> BS_LOG2) << BS_LOG2
        e1_hi = jnp.minimum(a_row, hi)
        e2_lo = jnp.maximum(b_row, e1_hi)
        lead_cp[j].wait()
        trail_cp[j].wait()
        ml = (riota_l >= lo - lead_base) & (riota_l < e1_hi - lead_base)
        mt = (riota_t >= e2_lo - b_row) & (riota_t < hi - b_row)
        out_ref[j, :] = (jnp.sum(jnp.where(ml, lead_buf[j], 0.0), axis=0)
                         + jnp.sum(jnp.where(mt, trail_buf[j], 0.0), axis=0))


def _edges_tc(starts, ends, feats):
    return pl.pallas_call(
        _edges_tc_body,
        out_shape=jax.ShapeDtypeStruct((NSEG, D), jnp.float32),
        in_specs=[
            pl.BlockSpec(memory_space=pltpu.SMEM),
            pl.BlockSpec(memory_space=pltpu.SMEM),
            pl.BlockSpec(memory_space=pltpu.HBM),
        ],
        out_specs=pl.BlockSpec(memory_space=pltpu.VMEM),
        scratch_shapes=[
            pltpu.VMEM((NSEG, EW, D), jnp.float32),
            pltpu.VMEM((NSEG, BS, D), jnp.float32),
            pltpu.SemaphoreType.DMA((NSEG,)),
            pltpu.SemaphoreType.DMA((NSEG,)),
        ],
    )(starts, ends, feats)


def _final_tc_body(starts_smem, ends_smem, blocks_ref, edges_ref, out_ref):
    blk_full = jnp.concatenate([blocks_ref[0], blocks_ref[1]], axis=1)
    biota = lax.broadcasted_iota(jnp.int32, (1, NBLK), 1)
    rows = []
    for j in range(NSEG):
        fb = (starts_smem[j] + BS - 1) >> BS_LOG2
        lb = ends_smem[j] >> BS_LOG2
        rows.append(jnp.where((biota >= fb) & (biota < lb), 1.0, 0.0))
    mask = jnp.concatenate(rows, axis=0)
    part = jax.lax.dot_general(
        mask, blk_full, (((1,), (0,)), ((), ())),
        precision=jax.lax.Precision.HIGHEST,
        preferred_element_type=jnp.float32)
    for j in range(NSEG):
        inv_n = 1.0 / jnp.maximum(ends_smem[j] - starts_smem[j], 1).astype(jnp.float32)
        out_ref[j, :] = (part[j, :] + edges_ref[j, :]) * inv_n


def _final_tc(starts, ends, blocks, edges):
    return pl.pallas_call(
        _final_tc_body,
        out_shape=jax.ShapeDtypeStruct((NSEG, D), jnp.float32),
        in_specs=[
            pl.BlockSpec(memory_space=pltpu.SMEM),
            pl.BlockSpec(memory_space=pltpu.SMEM),
            pl.BlockSpec(memory_space=pltpu.VMEM),
            pl.BlockSpec(memory_space=pltpu.VMEM),
        ],
        out_specs=pl.BlockSpec(memory_space=pltpu.VMEM),
    )(starts, ends, blocks, edges)


def kernel(inputs, offsets):
    feats = inputs
    offs = offsets.astype(jnp.int32)
    blocks = _block_sums(feats, offs)
    edges = _edges_tc(offs[:NSEG], offs[1:], feats)
    return _final_tc(offs[:NSEG], offs[1:], blocks, edges)


# speedup vs baseline: 9.1022x; 1.0492x over previous
import functools

import jax
import jax.numpy as jnp
from jax import lax
from jax.experimental import pallas as pl
from jax.experimental.pallas import tpu as pltpu
from jax.experimental.pallas import tpu_sc as plsc

N_TOK = 32768
D = 256
NSEG = 16
L = 16
NW = 32
BS = 128
BS_LOG2 = 7
NBLK = N_TOK // BS
BPW = NBLK // NW
RW = N_TOK // NW
DC = D // L
H = D // 2
HC = H // L
EW = BS + 8

_MESH = plsc.VectorSubcoreMesh(core_axis_name="c", subcore_axis_name="s")


@functools.partial(
    pl.kernel,
    mesh=_MESH,
    out_type=jax.ShapeDtypeStruct((2 * NBLK * H,), jnp.float32),
    scratch_types=[
        pltpu.VMEM((2, BS, D), jnp.float32),
        pltpu.VMEM((BPW, H), jnp.float32),
        pltpu.VMEM((BPW, H), jnp.float32),
        pltpu.VMEM((2 * L,), jnp.int32),
        pltpu.SemaphoreType.DMA,
        pltpu.SemaphoreType.DMA,
        pltpu.SemaphoreType.DMA,
    ],
)
def _block_sums(feats_hbm, offs_hbm, blocks_hbm, buf, bsum_lo, bsum_hi,
                offs_v, sem0, sem1, wsem):
    wid = lax.axis_index("c") * 16 + lax.axis_index("s")
    sems = (sem0, sem1)

    pltpu.sync_copy(offs_hbm, offs_v.at[pl.ds(0, NSEG + 1)])
    first = offs_v[pl.ds(0, L)][0]
    last = offs_v[pl.ds(NSEG, L)][0]
    b_lo = first >> BS_LOG2
    b_hi = (last + BS - 1) >> BS_LOG2
    nlive = b_hi - b_lo
    kper = (nlive + NW - 1) >> 5
    kw = jnp.clip(nlive - wid * kper, 0, kper)

    def start(i, s):
        blk = b_lo + wid * kper + i
        pltpu.async_copy(
            feats_hbm.at[pl.ds(blk * BS, BS), :], buf.at[s], sems[s])

    @pl.when(kw > 0)
    def _():
        start(0, 0)

    @pl.when(kw > 1)
    def _():
        start(1, 1)

    def outer(t, carry):
        for s in range(2):
            i = 2 * t + s

            @pl.when(i < kw)
            def _(i=i, s=s):
                pltpu.make_async_copy(
                    feats_hbm.at[pl.ds(0, BS), :], buf.at[s], sems[s]
                ).wait()

                def body(r, acc):
                    return tuple(
                        acc[k] + buf[s, r, pl.ds(k * L, L)] for k in range(DC)
                    )

                acc = lax.fori_loop(
                    0, BS, body,
                    tuple(jnp.zeros((L,), jnp.float32) for _ in range(DC)),
                )
                for k in range(HC):
                    bsum_lo[i, pl.ds(k * L, L)] = acc[k]
                    bsum_hi[i, pl.ds(k * L, L)] = acc[HC + k]
                blk = b_lo + wid * kper + i
                pltpu.async_copy(
                    bsum_lo.at[i], blocks_hbm.at[pl.ds(blk * H, H)], wsem)
                pltpu.async_copy(
                    bsum_hi.at[i],
                    blocks_hbm.at[pl.ds((NBLK + blk) * H, H)], wsem)

                @pl.when(i + 2 < kw)
                def _():
                    start(i + 2, s)
        return carry

    lax.fori_loop(0, (kw + 1) >> 1, outer, 0)

    def drain(_, carry):
        pltpu.make_async_copy(
            bsum_lo.at[0], blocks_hbm.at[pl.ds(0, H)], wsem).wait()
        return carry

    lax.fori_loop(0, 2 * kw, drain, 0)


def _edges_tc_body(starts_smem, ends_smem, feats_ref, out_ref,
                   lead_buf, trail_buf, lead_sem, trail_sem):
    lead_cp = [None] * NSEG
    trail_cp = [None] * NSEG
    for j in range(NSEG):
        lo = starts_smem[j]
        hi = ends_smem[j]
        lead_base = pl.multiple_of(jnp.minimum(lo & ~7, N_TOK - EW), 8)
        trail_base = pl.multiple_of((hi >> BS_LOG2) << BS_LOG2, BS)
        lead_cp[j] = pltpu.make_async_copy(
            feats_ref.at[pl.ds(lead_base, EW), :], lead_buf.at[j], lead_sem.at[j])
        trail_cp[j] = pltpu.make_async_copy(
            feats_ref.at[pl.ds(trail_base, BS), :], trail_buf.at[j], trail_sem.at[j])
        lead_cp[j].start()
        trail_cp[j].start()

    riota_l = lax.broadcasted_iota(jnp.int32, (EW, 1), 0)
    riota_t = lax.broadcasted_iota(jnp.int32, (BS, 1), 0)
    for j in range(NSEG):
        lo = starts_smem[j]
        hi = ends_smem[j]
        lead_base = jnp.minimum(lo & ~7, N_TOK - EW)
        a_row = ((lo + BS - 1) >> BS_LOG2) << BS_LOG2
        b_row = (hi >> BS_LOG2) << BS_LOG2
        e1_hi = jnp.minimum(a_row, hi)
        e2_lo = jnp.maximum(b_row, e1_hi)
        lead_cp[j].wait()
        trail_cp[j].wait()
        ml = (riota_l >= lo - lead_base) & (riota_l < e1_hi - lead_base)
        mt = (riota_t >= e2_lo - b_row) & (riota_t < hi - b_row)
        out_ref[j, :] = (jnp.sum(jnp.where(ml, lead_buf[j], 0.0), axis=0)
                         + jnp.sum(jnp.where(mt, trail_buf[j], 0.0), axis=0))


def _edges_tc(starts, ends, feats):
    return pl.pallas_call(
        _edges_tc_body,
        out_shape=jax.ShapeDtypeStruct((NSEG, D), jnp.float32),
        in_specs=[
            pl.BlockSpec(memory_space=pltpu.SMEM),
            pl.BlockSpec(memory_space=pltpu.SMEM),
            pl.BlockSpec(memory_space=pltpu.HBM),
        ],
        out_specs=pl.BlockSpec(memory_space=pltpu.VMEM),
        scratch_shapes=[
            pltpu.VMEM((NSEG, EW, D), jnp.float32),
            pltpu.VMEM((NSEG, BS, D), jnp.float32),
            pltpu.SemaphoreType.DMA((NSEG,)),
            pltpu.SemaphoreType.DMA((NSEG,)),
        ],
    )(starts, ends, feats)


def _final_tc_body(starts_smem, ends_smem, blocks_ref, edges_ref, out_ref):
    blk_full = jnp.concatenate([blocks_ref[0], blocks_ref[1]], axis=1)
    b_lo = starts_smem[0] >> BS_LOG2
    b_hi = (ends_smem[NSEG - 1] + BS - 1) >> BS_LOG2
    riota_b = lax.broadcasted_iota(jnp.int32, (NBLK, 1), 0)
    blk_full = jnp.where((riota_b >= b_lo) & (riota_b < b_hi), blk_full, 0.0)
    biota = lax.broadcasted_iota(jnp.int32, (1, NBLK), 1)
    rows = []
    for j in range(NSEG):
        fb = (starts_smem[j] + BS - 1) >> BS_LOG2
        lb = ends_smem[j] >> BS_LOG2
        rows.append(jnp.where((biota >= fb) & (biota < lb), 1.0, 0.0))
    mask = jnp.concatenate(rows, axis=0)
    part = jax.lax.dot_general(
        mask, blk_full, (((1,), (0,)), ((), ())),
        precision=jax.lax.Precision.HIGHEST,
        preferred_element_type=jnp.float32)
    for j in range(NSEG):
        inv_n = 1.0 / jnp.maximum(ends_smem[j] - starts_smem[j], 1).astype(jnp.float32)
        out_ref[j, :] = (part[j, :] + edges_ref[j, :]) * inv_n


def _final_tc(starts, ends, blocks, edges):
    return pl.pallas_call(
        _final_tc_body,
        out_shape=jax.ShapeDtypeStruct((NSEG, D), jnp.float32),
        in_specs=[
            pl.BlockSpec(memory_space=pltpu.SMEM),
            pl.BlockSpec(memory_space=pltpu.SMEM),
            pl.BlockSpec(memory_space=pltpu.VMEM),
            pl.BlockSpec(memory_space=pltpu.VMEM),
        ],
        out_specs=pl.BlockSpec(memory_space=pltpu.VMEM),
    )(starts, ends, blocks, edges)


def kernel(inputs, offsets):
    feats = inputs
    offs = offsets.astype(jnp.int32)
    blocks = _block_sums(feats, offs).reshape(2, NBLK, H)
    edges = _edges_tc(offs[:NSEG], offs[1:], feats)
    return _final_tc(offs[:NSEG], offs[1:], blocks, edges)


# speedup vs baseline: 9.6323x; 1.0582x over previous
import functools

import jax
import jax.numpy as jnp
from jax import lax
from jax.experimental import pallas as pl
from jax.experimental.pallas import tpu as pltpu
from jax.experimental.pallas import tpu_sc as plsc

N_TOK = 32768
D = 256
NSEG = 16
L = 16
NW = 32
BS = 128
BS_LOG2 = 7
NBLK = N_TOK // BS
BPW = NBLK // NW
RW = N_TOK // NW
DC = D // L
H = D // 2
HC = H // L
EW = BS + 8
TB = 112
TGRP = 8

_MESH = plsc.VectorSubcoreMesh(core_axis_name="c", subcore_axis_name="s")


@functools.partial(
    pl.kernel,
    mesh=_MESH,
    out_type=jax.ShapeDtypeStruct((2 * NBLK * H,), jnp.float32),
    scratch_types=[
        pltpu.VMEM((2, BS, D), jnp.float32),
        pltpu.VMEM((BPW, H), jnp.float32),
        pltpu.VMEM((BPW, H), jnp.float32),
        pltpu.VMEM((2 * L,), jnp.int32),
        pltpu.SemaphoreType.DMA,
        pltpu.SemaphoreType.DMA,
        pltpu.SemaphoreType.DMA,
    ],
)
def _block_sums(feats_hbm, offs_hbm, blocks_hbm, buf, bsum_lo, bsum_hi,
                offs_v, sem0, sem1, wsem):
    wid = lax.axis_index("c") * 16 + lax.axis_index("s")
    sems = (sem0, sem1)

    pltpu.sync_copy(offs_hbm, offs_v.at[pl.ds(0, NSEG + 1)])
    first = offs_v[pl.ds(0, L)][0]
    last = offs_v[pl.ds(NSEG, L)][0]
    b_lo = jnp.maximum(first >> BS_LOG2, TB)
    b_hi = (last + BS - 1) >> BS_LOG2
    nlive = jnp.maximum(b_hi - b_lo, 0)
    kper = (nlive + NW - 1) >> 5
    kw = jnp.clip(nlive - wid * kper, 0, kper)

    def start(i, s):
        blk = b_lo + wid * kper + i
        pltpu.async_copy(
            feats_hbm.at[pl.ds(blk * BS, BS), :], buf.at[s], sems[s])

    @pl.when(kw > 0)
    def _():
        start(0, 0)

    @pl.when(kw > 1)
    def _():
        start(1, 1)

    def outer(t, carry):
        for s in range(2):
            i = 2 * t + s

            @pl.when(i < kw)
            def _(i=i, s=s):
                pltpu.make_async_copy(
                    feats_hbm.at[pl.ds(0, BS), :], buf.at[s], sems[s]
                ).wait()

                def body(r, acc):
                    return tuple(
                        acc[k] + buf[s, r, pl.ds(k * L, L)] for k in range(DC)
                    )

                acc = lax.fori_loop(
                    0, BS, body,
                    tuple(jnp.zeros((L,), jnp.float32) for _ in range(DC)),
                )
                for k in range(HC):
                    bsum_lo[i, pl.ds(k * L, L)] = acc[k]
                    bsum_hi[i, pl.ds(k * L, L)] = acc[HC + k]
                blk = b_lo + wid * kper + i
                pltpu.async_copy(
                    bsum_lo.at[i], blocks_hbm.at[pl.ds(blk * H, H)], wsem)
                pltpu.async_copy(
                    bsum_hi.at[i],
                    blocks_hbm.at[pl.ds((NBLK + blk) * H, H)], wsem)

                @pl.when(i + 2 < kw)
                def _():
                    start(i + 2, s)
        return carry

    lax.fori_loop(0, (kw + 1) >> 1, outer, 0)

    def drain(_, carry):
        pltpu.make_async_copy(
            bsum_lo.at[0], blocks_hbm.at[pl.ds(0, H)], wsem).wait()
        return carry

    lax.fori_loop(0, 2 * kw, drain, 0)


def _edges_tc_body(starts_smem, ends_smem, feats_ref, out_ref,
                   lead_buf, trail_buf, lead_sem, trail_sem):
    lead_cp = [None] * NSEG
    trail_cp = [None] * NSEG
    for j in range(NSEG):
        lo = starts_smem[j]
        hi = ends_smem[j]
        lead_base = pl.multiple_of(jnp.minimum(lo & ~7, N_TOK - EW), 8)
        trail_base = pl.multiple_of((hi >> BS_LOG2) << BS_LOG2, BS)
        lead_cp[j] = pltpu.make_async_copy(
            feats_ref.at[pl.ds(lead_base, EW), :], lead_buf.at[j], lead_sem.at[j])
        trail_cp[j] = pltpu.make_async_copy(
            feats_ref.at[pl.ds(trail_base, BS), :], trail_buf.at[j], trail_sem.at[j])
        lead_cp[j].start()
        trail_cp[j].start()

    riota_l = lax.broadcasted_iota(jnp.int32, (EW, 1), 0)
    riota_t = lax.broadcasted_iota(jnp.int32, (BS, 1), 0)
    for j in range(NSEG):
        lo = starts_smem[j]
        hi = ends_smem[j]
        lead_base = jnp.minimum(lo & ~7, N_TOK - EW)
        a_row = ((lo + BS - 1) >> BS_LOG2) << BS_LOG2
        b_row = (hi >> BS_LOG2) << BS_LOG2
        e1_hi = jnp.minimum(a_row, hi)
        e2_lo = jnp.maximum(b_row, e1_hi)
        lead_cp[j].wait()
        trail_cp[j].wait()
        ml = (riota_l >= lo - lead_base) & (riota_l < e1_hi - lead_base)
        mt = (riota_t >= e2_lo - b_row) & (riota_t < hi - b_row)
        out_ref[j, :] = (jnp.sum(jnp.where(ml, lead_buf[j], 0.0), axis=0)
                         + jnp.sum(jnp.where(mt, trail_buf[j], 0.0), axis=0))


def _edges_tc(starts, ends, feats):
    return pl.pallas_call(
        _edges_tc_body,
        out_shape=jax.ShapeDtypeStruct((NSEG, D), jnp.float32),
        in_specs=[
            pl.BlockSpec(memory_space=pltpu.SMEM),
            pl.BlockSpec(memory_space=pltpu.SMEM),
            pl.BlockSpec(memory_space=pltpu.HBM),
        ],
        out_specs=pl.BlockSpec(memory_space=pltpu.VMEM),
        scratch_shapes=[
            pltpu.VMEM((NSEG, EW, D), jnp.float32),
            pltpu.VMEM((NSEG, BS, D), jnp.float32),
            pltpu.SemaphoreType.DMA((NSEG,)),
            pltpu.SemaphoreType.DMA((NSEG,)),
        ],
    )(starts, ends, feats)


def _tc_block_sums_body(feats_ref, out_ref):
    x = feats_ref[...]
    out_ref[...] = jnp.sum(x.reshape(TGRP, BS, D), axis=1)


def _tc_block_sums(feats):
    return pl.pallas_call(
        _tc_block_sums_body,
        grid=(TB // TGRP,),
        in_specs=[
            pl.BlockSpec((TGRP * BS, D), lambda i: (i, 0)),
        ],
        out_specs=pl.BlockSpec((TGRP, D), lambda i: (i, 0)),
        out_shape=jax.ShapeDtypeStruct((TB, D), jnp.float32),
    )(feats)


def _final_tc_body(starts_smem, ends_smem, blocks_ref, tcblocks_ref, edges_ref,
                   out_ref):
    sc_full = jnp.concatenate([blocks_ref[0], blocks_ref[1]], axis=1)
    blk_full = jnp.concatenate([tcblocks_ref[...], sc_full[TB:, :]], axis=0)
    b_lo = starts_smem[0] >> BS_LOG2
    b_hi = (ends_smem[NSEG - 1] + BS - 1) >> BS_LOG2
    riota_b = lax.broadcasted_iota(jnp.int32, (NBLK, 1), 0)
    blk_full = jnp.where((riota_b >= b_lo) & (riota_b < b_hi), blk_full, 0.0)
    biota = lax.broadcasted_iota(jnp.int32, (1, NBLK), 1)
    rows = []
    for j in range(NSEG):
        fb = (starts_smem[j] + BS - 1) >> BS_LOG2
        lb = ends_smem[j] >> BS_LOG2
        rows.append(jnp.where((biota >= fb) & (biota < lb), 1.0, 0.0))
    mask = jnp.concatenate(rows, axis=0)
    part = jax.lax.dot_general(
        mask, blk_full, (((1,), (0,)), ((), ())),
        precision=jax.lax.Precision.HIGHEST,
        preferred_element_type=jnp.float32)
    for j in range(NSEG):
        inv_n = 1.0 / jnp.maximum(ends_smem[j] - starts_smem[j], 1).astype(jnp.float32)
        out_ref[j, :] = (part[j, :] + edges_ref[j, :]) * inv_n


def _final_tc(starts, ends, blocks, tcblocks, edges):
    return pl.pallas_call(
        _final_tc_body,
        out_shape=jax.ShapeDtypeStruct((NSEG, D), jnp.float32),
        in_specs=[
            pl.BlockSpec(memory_space=pltpu.SMEM),
            pl.BlockSpec(memory_space=pltpu.SMEM),
            pl.BlockSpec(memory_space=pltpu.VMEM),
            pl.BlockSpec(memory_space=pltpu.VMEM),
            pl.BlockSpec(memory_space=pltpu.VMEM),
        ],
        out_specs=pl.BlockSpec(memory_space=pltpu.VMEM),
    )(starts, ends, blocks, tcblocks, edges)


def kernel(inputs, offsets):
    feats = inputs
    offs = offsets.astype(jnp.int32)
    blocks = _block_sums(feats, offs).reshape(2, NBLK, H)
    tcblocks = _tc_block_sums(feats)
    edges = _edges_tc(offs[:NSEG], offs[1:], feats)
    return _final_tc(offs[:NSEG], offs[1:], blocks, tcblocks, edges)
